# baseline (speedup 1.0000x reference)
import jax
import jax.numpy as jnp
from jax import lax
from jax.experimental import pallas as pl
from jax.experimental.pallas import tpu as pltpu

_TILE = 256
_PREP_BLK = 1024


def _round_up(n, m):
    return ((n + m - 1) // m) * m


def _prep_body(g0_ref, m0_ref, g1_ref, m1_ref, t0_ref, t1_ref):
    blk, d = g0_ref.shape
    lane = lax.broadcasted_iota(jnp.int32, (blk, d), 1)
    low = lane < d // 2
    for g_ref, m_ref, t_ref in ((g0_ref, m0_ref, t0_ref),
                                (g1_ref, m1_ref, t1_ref)):
        g = g_ref[...]
        m = m_ref[...]
        t_ref[:, 0, :] = jnp.where(low, g, pltpu.roll(m, d // 2, axis=1))
        t_ref[:, 1, :] = jnp.where(low, pltpu.roll(g, d // 2, axis=1), m)


def _build_tables(gmf_t0, gmf_t1, mlp_t0, mlp_t1):
    V, E = gmf_t0.shape
    D = 2 * E
    half = V // 2
    ins = [a.reshape(half, D) for a in (gmf_t0, mlp_t0, gmf_t1, mlp_t1)]
    blk = min(_PREP_BLK, half)
    blk_in = pl.BlockSpec((blk, D), lambda b: (b, 0))
    blk_out = pl.BlockSpec((blk, 2, D), lambda b: (b, 0, 0))
    t0, t1 = pl.pallas_call(
        _prep_body,
        out_shape=[jax.ShapeDtypeStruct((half, 2, D), jnp.float32)] * 2,
        grid=(half // blk,),
        in_specs=[blk_in] * 4,
        out_specs=[blk_out] * 2,
        compiler_params=pltpu.CompilerParams(
            dimension_semantics=("arbitrary",)),
    )(*ins)
    return t0.reshape(V, 1, D), t1.reshape(V, 1, D)


def _ncf_body(idx_ref,
              t0_ref, t1_ref,
              w1a_ref, w1b_ref,
              b1_ref, w2_ref, b2_ref,
              wg_ref, wm_ref,
              bfc_ref,
              out_ref,
              a0, a1):
    for m in range(_TILE):
        a0[m] = t0_ref[idx_ref[m, 0], 0]
        a1[m] = t1_ref[idx_ref[m, 1], 0]

    A0 = a0[...]
    A1 = a1[...]
    prod = A0 * A1

    h = (jnp.dot(A0, w1a_ref[...], preferred_element_type=jnp.float32)
         + jnp.dot(A1, w1b_ref[...], preferred_element_type=jnp.float32)
         + b1_ref[...])
    h = jnp.maximum(h, 0.0)
    h = jnp.dot(h, w2_ref[...], preferred_element_type=jnp.float32) + b2_ref[...]
    h = jnp.maximum(h, 0.0)

    logit = (jnp.sum(prod * wg_ref[...], axis=-1, keepdims=True)
             + jnp.sum(h * wm_ref[...], axis=-1, keepdims=True)
             + bfc_ref[0, 0])
    out_ref[...] = jax.nn.sigmoid(logit)


def kernel(x, gmf_t0, gmf_t1, mlp_t0, mlp_t1, w1, b1, w2, b2, wfc, bfc):
    B = x.shape[0]
    E = gmf_t0.shape[1]
    D = 2 * E

    b_pad = _round_up(max(B, 1), _TILE)
    num_tiles = b_pad // _TILE

    idx = x.astype(jnp.int32)
    if b_pad != B:
        idx = jnp.pad(idx, ((0, b_pad - B), (0, 0)))

    t0, t1 = _build_tables(gmf_t0, gmf_t1, mlp_t0, mlp_t1)

    zeros_top = jnp.zeros((E, 128), jnp.float32)
    w1a = jnp.concatenate([zeros_top, w1[:E, :]], axis=0)
    w1b = jnp.concatenate([zeros_top, w1[E:, :]], axis=0)
    wg = jnp.pad(wfc[:E, :].T, ((0, 0), (0, D - E)))
    wm = wfc[E:, :].T

    def resident(a):
        return pl.BlockSpec(a.shape, lambda g: (0,) * a.ndim)

    flops = 2 * b_pad * (D * 128 * 2 + 128 * 64) + b_pad * (4 * D + 4 * 64)
    bytes_accessed = (t0.size + t1.size) * 4 + b_pad * (2 * 4 + D * 8 + 4)
    out = pl.pallas_call(
        _ncf_body,
        out_shape=jax.ShapeDtypeStruct((b_pad, 1), jnp.float32),
        grid=(num_tiles,),
        in_specs=[
            pl.BlockSpec((_TILE, 2), lambda g: (g, 0),
                         memory_space=pltpu.MemorySpace.SMEM),
            resident(t0), resident(t1),
            resident(w1a), resident(w1b), resident(b1),
            resident(w2), resident(b2),
            resident(wg), resident(wm),
            pl.BlockSpec(memory_space=pltpu.MemorySpace.SMEM),
        ],
        out_specs=pl.BlockSpec((_TILE, 1), lambda g: (g, 0)),
        scratch_shapes=[
            pltpu.VMEM((_TILE, D), jnp.float32),
            pltpu.VMEM((_TILE, D), jnp.float32),
        ],
        compiler_params=pltpu.CompilerParams(
            dimension_semantics=("parallel",)),
        cost_estimate=pl.CostEstimate(flops=flops, transcendentals=b_pad,
                                      bytes_accessed=bytes_accessed),
    )(idx, t0, t1, w1a, w1b, b1, w2, b2, wg, wm, bfc)
    return out[:B]
